# baseline (speedup 1.0000x reference)
import functools

import jax
import jax.numpy as jnp
from jax import lax
from jax.experimental import pallas as pl
from jax.experimental.pallas import tpu as pltpu
from jax.experimental.pallas import tpu_sc as plsc

D_MODEL = 64
SCALE = 8.0

_info = plsc.get_sparse_core_info()
_NC, _NS = _info.num_cores, _info.num_subcores
_NW = _NC * _NS

S1 = 128


def _make_gather(n_batch, seq_len):
    rows_per_w = n_batch // _NW
    s2 = seq_len - S1
    mesh = plsc.VectorSubcoreMesh(core_axis_name="c", subcore_axis_name="s")

    @functools.partial(
        pl.kernel,
        mesh=mesh,
        compiler_params=pltpu.CompilerParams(use_tc_tiling_on_sc=False),
        out_type=jax.ShapeDtypeStruct((n_batch, seq_len, D_MODEL), jnp.float32),
        scratch_types=[
            pltpu.VMEM((rows_per_w, seq_len), jnp.int32),
            pltpu.VMEM((2, seq_len, D_MODEL), jnp.float32),
            pltpu.SemaphoreType.DMA,
            pltpu.SemaphoreType.DMA,
            pltpu.SemaphoreType.DMA,
            pltpu.SemaphoreType.DMA,
        ],
    )
    def gather_scale(idx_hbm, table_hbm, out_hbm, idx_v, rows_v, g0, g1, w0, w1):
        wid = lax.axis_index("s") * _NC + lax.axis_index("c")
        row0 = wid * rows_per_w
        sem_g = [g0, g1]
        sem_w = [w0, w1]

        pltpu.sync_copy(idx_hbm.at[pl.ds(row0, rows_per_w)], idx_v)

        def fire_gathers(g, b):
            pltpu.async_copy(
                table_hbm.at[idx_v.at[g, pl.ds(0, S1)]],
                rows_v.at[b].at[pl.ds(0, S1)],
                sem_g[b],
            )
            pltpu.async_copy(
                table_hbm.at[idx_v.at[g, pl.ds(S1, s2)]],
                rows_v.at[b].at[pl.ds(S1, s2)],
                sem_g[b],
            )

        def drain(sem):
            pltpu.make_async_copy(
                table_hbm.at[pl.ds(0, seq_len)], rows_v.at[0], sem
            ).wait()

        fire_gathers(0, 0)

        def outer(go, carry):
            for b in range(2):
                g = 2 * go + b
                bn = 1 - b

                @pl.when(g < rows_per_w - 1)
                def _():
                    @pl.when(g >= 1)
                    def _():
                        drain(sem_w[bn])

                    fire_gathers(g + 1, bn)

                drain(sem_g[b])

                def mul_body(i, c2):
                    for j in range(D_MODEL // 16):
                        sl = pl.ds(j * 16, 16)
                        rows_v[b, i, sl] = rows_v[b, i, sl] * SCALE
                    return c2

                lax.fori_loop(0, seq_len, mul_body, 0, unroll=4)
                pltpu.async_copy(
                    rows_v.at[b],
                    out_hbm.at[row0 + g],
                    sem_w[b],
                )
            return carry

        lax.fori_loop(0, rows_per_w // 2, outer, 0)
        drain(sem_w[0])
        drain(sem_w[1])

    return gather_scale


def kernel(x, lut):
    b, t = x.shape
    idx = x.astype(jnp.int32)
    return _make_gather(b, t)(idx, lut)
